# baseline (speedup 1.0000x reference)
import jax
import jax.numpy as jnp
from jax.experimental import pallas as pl

_N_TOKENS = 16384
_INPUT_DIM = 256
_HIDDEN = 64
_K = 1024
_S = 3
_TN = 512


def _bdot(a, b):
    return jnp.dot(a.astype(jnp.bfloat16), b,
                   preferred_element_type=jnp.float32)


def _rowsum_sq(v):
    sq = v * v
    p = sq[:, 0:8]
    for j in range(1, 8):
        p = p + sq[:, 8 * j:8 * j + 8]
    s = [p[:, k:k + 1] for k in range(8)]
    return (((s[0] + s[1]) + (s[2] + s[3]))
            + ((s[4] + s[5]) + (s[6] + s[7])))


def _colsum_sq(v):
    sq = v * v
    p = sq[0:8, :]
    for j in range(1, 8):
        p = p + sq[8 * j:8 * j + 8, :]
    s = [p[k:k + 1, :] for k in range(8)]
    return (((s[0] + s[1]) + (s[2] + s[3]))
            + ((s[4] + s[5]) + (s[6] + s[7])))


def _rvq_body(x_ref, w1_ref, b1_ref, w2_ref, b2_ref, w3_ref, b3_ref,
              w4_ref, b4_ref, cbt_ref, cbtb_ref, hi_ref, mid_ref, lo_ref,
              q_ref, idx_ref, rec_ref):
    h1 = jnp.maximum(_bdot(x_ref[...], w1_ref[...]) + b1_ref[...], 0.0)
    h = _bdot(h1, w2_ref[...]) + b2_ref[...]

    r = h
    q = jnp.zeros_like(h)
    iota = jax.lax.broadcasted_iota(jnp.int32, (_TN, _K), 1)
    for s in range(_S):
        cn = _colsum_sq(cbt_ref[s])
        rn = _rowsum_sq(r)
        ab = _bdot(r, cbtb_ref[s])
        d = jnp.sqrt(jnp.maximum((rn + cn) - 2.0 * ab, 0.0))
        m = jnp.min(d, axis=1, keepdims=True)
        idx = jnp.min(jnp.where(d == m, iota, _K), axis=1, keepdims=True)
        idx_ref[:, s:s + 1] = idx
        oh = (iota == idx).astype(jnp.bfloat16)
        sel = (jnp.dot(oh, hi_ref[s], preferred_element_type=jnp.float32)
               + jnp.dot(oh, mid_ref[s], preferred_element_type=jnp.float32)
               + jnp.dot(oh, lo_ref[s], preferred_element_type=jnp.float32))
        q = q + sel
        r = r - sel

    q_ref[...] = q
    d1 = jnp.maximum(_bdot(q, w3_ref[...]) + b3_ref[...], 0.0)
    rec_ref[...] = _bdot(d1, w4_ref[...]) + b4_ref[...]


def kernel(x, W1, b1, W2, b2, W3, b3, W4, b4, codebooks):
    f32 = jnp.float32
    bf16 = jnp.bfloat16
    cbt = jnp.swapaxes(codebooks, 1, 2)
    cbtb = cbt.astype(bf16)
    hi = jax.lax.optimization_barrier(codebooks.astype(bf16))
    rem1 = codebooks - hi.astype(f32)
    mid = jax.lax.optimization_barrier(rem1.astype(bf16))
    lo = jax.lax.optimization_barrier(
        (rem1 - mid.astype(f32)).astype(bf16))

    grid = (_N_TOKENS // _TN,)
    full = lambda shape: pl.BlockSpec(shape, lambda i: (0,) * len(shape))
    q, idx, rec = pl.pallas_call(
        _rvq_body,
        grid=grid,
        in_specs=[
            pl.BlockSpec((_TN, _INPUT_DIM), lambda i: (i, 0)),
            full((_INPUT_DIM, 2 * _HIDDEN)),
            full((1, 2 * _HIDDEN)),
            full((2 * _HIDDEN, _HIDDEN)),
            full((1, _HIDDEN)),
            full((_HIDDEN, 2 * _HIDDEN)),
            full((1, 2 * _HIDDEN)),
            full((2 * _HIDDEN, _INPUT_DIM)),
            full((1, _INPUT_DIM)),
            full((_S, _HIDDEN, _K)),
            full((_S, _HIDDEN, _K)),
            full((_S, _K, _HIDDEN)),
            full((_S, _K, _HIDDEN)),
            full((_S, _K, _HIDDEN)),
        ],
        out_specs=[
            pl.BlockSpec((_TN, _HIDDEN), lambda i: (i, 0)),
            pl.BlockSpec((_TN, _S), lambda i: (i, 0)),
            pl.BlockSpec((_TN, _INPUT_DIM), lambda i: (i, 0)),
        ],
        out_shape=[
            jax.ShapeDtypeStruct((_N_TOKENS, _HIDDEN), f32),
            jax.ShapeDtypeStruct((_N_TOKENS, _S), jnp.int32),
            jax.ShapeDtypeStruct((_N_TOKENS, _INPUT_DIM), f32),
        ],
    )(x, W1.astype(bf16), b1.reshape(1, -1), W2.astype(bf16),
      b2.reshape(1, -1), W3.astype(bf16), b3.reshape(1, -1),
      W4.astype(bf16), b4.reshape(1, -1), cbt, cbtb, hi, mid, lo)
    return (q, idx.T, rec)


# speedup vs baseline: 1.2348x; 1.2348x over previous
import jax
import jax.numpy as jnp
from jax.experimental import pallas as pl

_N_TOKENS = 16384
_INPUT_DIM = 256
_HIDDEN = 64
_K = 1024
_S = 3
_TN = 512


def _bdot(a, b):
    return jnp.dot(a.astype(jnp.bfloat16), b,
                   preferred_element_type=jnp.float32)


def _rowsum_sq(v):
    sq = v * v
    p = sq[:, 0:8]
    for j in range(1, 8):
        p = p + sq[:, 8 * j:8 * j + 8]
    s = [p[:, k:k + 1] for k in range(8)]
    return (((s[0] + s[1]) + (s[2] + s[3]))
            + ((s[4] + s[5]) + (s[6] + s[7])))


def _colsum_sq(v):
    sq = v * v
    p = sq[0:8, :]
    for j in range(1, 8):
        p = p + sq[8 * j:8 * j + 8, :]
    s = [p[k:k + 1, :] for k in range(8)]
    return (((s[0] + s[1]) + (s[2] + s[3]))
            + ((s[4] + s[5]) + (s[6] + s[7])))


def _rvq_body(x_ref, w1_ref, b1_ref, w2_ref, b2_ref, w3_ref, b3_ref,
              w4_ref, b4_ref, cbt_ref, cbtb_ref, hi_ref, mid_ref, lo_ref,
              q_ref, idx_ref, rec_ref):
    h1 = jnp.maximum(_bdot(x_ref[...], w1_ref[...]) + b1_ref[...], 0.0)
    h = _bdot(h1, w2_ref[...]) + b2_ref[...]

    r = h
    q = jnp.zeros_like(h)
    iota = jax.lax.broadcasted_iota(jnp.int32, (_TN, _K), 1)
    for s in range(_S):
        cbt = cbt_ref[s]
        cn = jnp.sum(cbt * cbt, axis=0, keepdims=True)
        rn = jnp.sum(r * r, axis=1, keepdims=True)
        ab = _bdot(r, cbtb_ref[s])
        d = jnp.sqrt(jnp.maximum((rn + cn) - 2.0 * ab, 0.0))
        m = jnp.min(d, axis=1, keepdims=True)
        idx = jnp.min(jnp.where(d == m, iota, _K), axis=1, keepdims=True)
        idx_ref[:, s:s + 1] = idx
        oh = (iota == idx).astype(jnp.bfloat16)
        sel = (jnp.dot(oh, hi_ref[s], preferred_element_type=jnp.float32)
               + jnp.dot(oh, mid_ref[s], preferred_element_type=jnp.float32)
               + jnp.dot(oh, lo_ref[s], preferred_element_type=jnp.float32))
        q = q + sel
        r = r - sel

    q_ref[...] = q
    d1 = jnp.maximum(_bdot(q, w3_ref[...]) + b3_ref[...], 0.0)
    rec_ref[...] = _bdot(d1, w4_ref[...]) + b4_ref[...]


def kernel(x, W1, b1, W2, b2, W3, b3, W4, b4, codebooks):
    f32 = jnp.float32
    bf16 = jnp.bfloat16
    cbt = jnp.swapaxes(codebooks, 1, 2)
    cbtb = cbt.astype(bf16)
    hi = jax.lax.optimization_barrier(codebooks.astype(bf16))
    rem1 = codebooks - hi.astype(f32)
    mid = jax.lax.optimization_barrier(rem1.astype(bf16))
    lo = jax.lax.optimization_barrier(
        (rem1 - mid.astype(f32)).astype(bf16))

    grid = (_N_TOKENS // _TN,)
    full = lambda shape: pl.BlockSpec(shape, lambda i: (0,) * len(shape))
    q, idx, rec = pl.pallas_call(
        _rvq_body,
        grid=grid,
        in_specs=[
            pl.BlockSpec((_TN, _INPUT_DIM), lambda i: (i, 0)),
            full((_INPUT_DIM, 2 * _HIDDEN)),
            full((1, 2 * _HIDDEN)),
            full((2 * _HIDDEN, _HIDDEN)),
            full((1, _HIDDEN)),
            full((_HIDDEN, 2 * _HIDDEN)),
            full((1, 2 * _HIDDEN)),
            full((2 * _HIDDEN, _INPUT_DIM)),
            full((1, _INPUT_DIM)),
            full((_S, _HIDDEN, _K)),
            full((_S, _HIDDEN, _K)),
            full((_S, _K, _HIDDEN)),
            full((_S, _K, _HIDDEN)),
            full((_S, _K, _HIDDEN)),
        ],
        out_specs=[
            pl.BlockSpec((_TN, _HIDDEN), lambda i: (i, 0)),
            pl.BlockSpec((_TN, _S), lambda i: (i, 0)),
            pl.BlockSpec((_TN, _INPUT_DIM), lambda i: (i, 0)),
        ],
        out_shape=[
            jax.ShapeDtypeStruct((_N_TOKENS, _HIDDEN), f32),
            jax.ShapeDtypeStruct((_N_TOKENS, _S), jnp.int32),
            jax.ShapeDtypeStruct((_N_TOKENS, _INPUT_DIM), f32),
        ],
    )(x, W1.astype(bf16), b1.reshape(1, -1), W2.astype(bf16),
      b2.reshape(1, -1), W3.astype(bf16), b3.reshape(1, -1),
      W4.astype(bf16), b4.reshape(1, -1), cbt, cbtb, hi, mid, lo)
    return (q, idx.T, rec)


# speedup vs baseline: 1.2575x; 1.0184x over previous
import jax
import jax.numpy as jnp
from jax.experimental import pallas as pl

_N_TOKENS = 16384
_INPUT_DIM = 256
_HIDDEN = 64
_K = 1024
_S = 3
_TN = 512


def _bdot(a, b):
    return jnp.dot(a.astype(jnp.bfloat16), b,
                   preferred_element_type=jnp.float32)


def _rowsum_sq(v):
    sq = v * v
    p = sq[:, 0:8]
    for j in range(1, 8):
        p = p + sq[:, 8 * j:8 * j + 8]
    s = [p[:, k:k + 1] for k in range(8)]
    return (((s[0] + s[1]) + (s[2] + s[3]))
            + ((s[4] + s[5]) + (s[6] + s[7])))


def _colsum_sq(v):
    sq = v * v
    p = sq[0:8, :]
    for j in range(1, 8):
        p = p + sq[8 * j:8 * j + 8, :]
    s = [p[k:k + 1, :] for k in range(8)]
    return (((s[0] + s[1]) + (s[2] + s[3]))
            + ((s[4] + s[5]) + (s[6] + s[7])))


def _rvq_body(x_ref, w1_ref, b1_ref, w2_ref, b2_ref, w3_ref, b3_ref,
              w4_ref, b4_ref, cbt_ref, cbtb_ref, hi_ref, mid_ref, lo_ref,
              q_ref, idx_ref, rec_ref):
    h1 = jnp.maximum(_bdot(x_ref[...], w1_ref[...]) + b1_ref[...], 0.0)
    h = _bdot(h1, w2_ref[...]) + b2_ref[...]

    r = h
    q = jnp.zeros_like(h)
    iota = jax.lax.broadcasted_iota(jnp.int32, (_TN, _K), 1)
    for s in range(_S):
        cbt = cbt_ref[s]
        cn = jnp.sum(cbt * cbt, axis=0, keepdims=True)
        rn = jnp.sum(r * r, axis=1, keepdims=True)
        ab2 = _bdot(r, cbtb_ref[s])
        d = jnp.sqrt(jnp.maximum((rn + cn) + ab2, 0.0))
        m = jnp.min(d, axis=1, keepdims=True)
        idx = jnp.min(jnp.where(d == m, iota, _K), axis=1, keepdims=True)
        idx_ref[:, s:s + 1] = idx
        oh = (iota == idx).astype(jnp.bfloat16)
        sel = (jnp.dot(oh, hi_ref[s], preferred_element_type=jnp.float32)
               + jnp.dot(oh, mid_ref[s], preferred_element_type=jnp.float32)
               + jnp.dot(oh, lo_ref[s], preferred_element_type=jnp.float32))
        q = q + sel
        r = r - sel

    q_ref[...] = q
    d1 = jnp.maximum(_bdot(q, w3_ref[...]) + b3_ref[...], 0.0)
    rec_ref[...] = _bdot(d1, w4_ref[...]) + b4_ref[...]


def kernel(x, W1, b1, W2, b2, W3, b3, W4, b4, codebooks):
    f32 = jnp.float32
    bf16 = jnp.bfloat16
    cbt = jnp.swapaxes(codebooks, 1, 2)
    cbtb = -2.0 * cbt.astype(bf16)
    hi = jax.lax.optimization_barrier(codebooks.astype(bf16))
    rem1 = codebooks - hi.astype(f32)
    mid = jax.lax.optimization_barrier(rem1.astype(bf16))
    lo = jax.lax.optimization_barrier(
        (rem1 - mid.astype(f32)).astype(bf16))

    grid = (_N_TOKENS // _TN,)
    full = lambda shape: pl.BlockSpec(shape, lambda i: (0,) * len(shape))
    q, idx, rec = pl.pallas_call(
        _rvq_body,
        grid=grid,
        in_specs=[
            pl.BlockSpec((_TN, _INPUT_DIM), lambda i: (i, 0)),
            full((_INPUT_DIM, 2 * _HIDDEN)),
            full((1, 2 * _HIDDEN)),
            full((2 * _HIDDEN, _HIDDEN)),
            full((1, _HIDDEN)),
            full((_HIDDEN, 2 * _HIDDEN)),
            full((1, 2 * _HIDDEN)),
            full((2 * _HIDDEN, _INPUT_DIM)),
            full((1, _INPUT_DIM)),
            full((_S, _HIDDEN, _K)),
            full((_S, _HIDDEN, _K)),
            full((_S, _K, _HIDDEN)),
            full((_S, _K, _HIDDEN)),
            full((_S, _K, _HIDDEN)),
        ],
        out_specs=[
            pl.BlockSpec((_TN, _HIDDEN), lambda i: (i, 0)),
            pl.BlockSpec((_TN, _S), lambda i: (i, 0)),
            pl.BlockSpec((_TN, _INPUT_DIM), lambda i: (i, 0)),
        ],
        out_shape=[
            jax.ShapeDtypeStruct((_N_TOKENS, _HIDDEN), f32),
            jax.ShapeDtypeStruct((_N_TOKENS, _S), jnp.int32),
            jax.ShapeDtypeStruct((_N_TOKENS, _INPUT_DIM), f32),
        ],
    )(x, W1.astype(bf16), b1.reshape(1, -1), W2.astype(bf16),
      b2.reshape(1, -1), W3.astype(bf16), b3.reshape(1, -1),
      W4.astype(bf16), b4.reshape(1, -1), cbt, cbtb, hi, mid, lo)
    return (q, idx.T, rec)


# speedup vs baseline: 1.5803x; 1.2567x over previous
import jax
import jax.numpy as jnp
from jax.experimental import pallas as pl

_N_TOKENS = 16384
_INPUT_DIM = 256
_HIDDEN = 64
_K = 1024
_S = 3
_TN = 512


def _bdot(a, b):
    return jnp.dot(a.astype(jnp.bfloat16), b,
                   preferred_element_type=jnp.float32)


def _rowsum_sq(v):
    sq = v * v
    p = sq[:, 0:8]
    for j in range(1, 8):
        p = p + sq[:, 8 * j:8 * j + 8]
    s = [p[:, k:k + 1] for k in range(8)]
    return (((s[0] + s[1]) + (s[2] + s[3]))
            + ((s[4] + s[5]) + (s[6] + s[7])))


def _colsum_sq(v):
    sq = v * v
    p = sq[0:8, :]
    for j in range(1, 8):
        p = p + sq[8 * j:8 * j + 8, :]
    s = [p[k:k + 1, :] for k in range(8)]
    return (((s[0] + s[1]) + (s[2] + s[3]))
            + ((s[4] + s[5]) + (s[6] + s[7])))


def _rvq_body(x_ref, w1_ref, b1_ref, w2_ref, b2_ref, w3_ref, b3_ref,
              w4_ref, b4_ref, cbt_ref, cbtb_ref, hml_ref,
              q_ref, idx_ref, rec_ref):
    h1 = jnp.maximum(_bdot(x_ref[...], w1_ref[...]) + b1_ref[...], 0.0)
    h = _bdot(h1, w2_ref[...]) + b2_ref[...]

    r = h
    q = jnp.zeros_like(h)
    iota = jax.lax.broadcasted_iota(jnp.int32, (_TN, _K), 1)
    for s in range(_S):
        cbt = cbt_ref[s]
        cn = jnp.sum(cbt * cbt, axis=0, keepdims=True)
        rn = jnp.sum(r * r, axis=1, keepdims=True)
        ab2 = _bdot(r, cbtb_ref[s])
        d = jnp.sqrt(jnp.maximum((rn + cn) + ab2, 0.0))
        m = jnp.min(d, axis=1, keepdims=True)
        idx = jnp.min(jnp.where(d == m, iota, _K), axis=1, keepdims=True)
        idx_ref[:, s:s + 1] = idx
        oh = (iota == idx).astype(jnp.bfloat16)
        p = jnp.dot(oh, hml_ref[s], preferred_element_type=jnp.float32)
        sel = (p[:, 0:_HIDDEN] + p[:, _HIDDEN:2 * _HIDDEN]) \
            + p[:, 2 * _HIDDEN:3 * _HIDDEN]
        q = q + sel
        r = r - sel

    q_ref[...] = q
    d1 = jnp.maximum(_bdot(q, w3_ref[...]) + b3_ref[...], 0.0)
    rec_ref[...] = _bdot(d1, w4_ref[...]) + b4_ref[...]


def kernel(x, W1, b1, W2, b2, W3, b3, W4, b4, codebooks):
    f32 = jnp.float32
    bf16 = jnp.bfloat16
    cbt = jnp.swapaxes(codebooks, 1, 2)
    cbtb = -2.0 * cbt.astype(bf16)
    hi = jax.lax.optimization_barrier(codebooks.astype(bf16))
    rem1 = codebooks - hi.astype(f32)
    mid = jax.lax.optimization_barrier(rem1.astype(bf16))
    lo = jax.lax.optimization_barrier(
        (rem1 - mid.astype(f32)).astype(bf16))
    hml = jnp.concatenate([hi, mid, lo], axis=2)

    grid = (_N_TOKENS // _TN,)
    full = lambda shape: pl.BlockSpec(shape, lambda i: (0,) * len(shape))
    q, idx, rec = pl.pallas_call(
        _rvq_body,
        grid=grid,
        in_specs=[
            pl.BlockSpec((_TN, _INPUT_DIM), lambda i: (i, 0)),
            full((_INPUT_DIM, 2 * _HIDDEN)),
            full((1, 2 * _HIDDEN)),
            full((2 * _HIDDEN, _HIDDEN)),
            full((1, _HIDDEN)),
            full((_HIDDEN, 2 * _HIDDEN)),
            full((1, 2 * _HIDDEN)),
            full((2 * _HIDDEN, _INPUT_DIM)),
            full((1, _INPUT_DIM)),
            full((_S, _HIDDEN, _K)),
            full((_S, _HIDDEN, _K)),
            full((_S, _K, 3 * _HIDDEN)),
        ],
        out_specs=[
            pl.BlockSpec((_TN, _HIDDEN), lambda i: (i, 0)),
            pl.BlockSpec((_TN, _S), lambda i: (i, 0)),
            pl.BlockSpec((_TN, _INPUT_DIM), lambda i: (i, 0)),
        ],
        out_shape=[
            jax.ShapeDtypeStruct((_N_TOKENS, _HIDDEN), f32),
            jax.ShapeDtypeStruct((_N_TOKENS, _S), jnp.int32),
            jax.ShapeDtypeStruct((_N_TOKENS, _INPUT_DIM), f32),
        ],
    )(x.astype(bf16), W1.astype(bf16), b1.reshape(1, -1), W2.astype(bf16),
      b2.reshape(1, -1), W3.astype(bf16), b3.reshape(1, -1),
      W4.astype(bf16), b4.reshape(1, -1), cbt, cbtb, hml)
    return (q, idx.T, rec)


# speedup vs baseline: 1.7155x; 1.0855x over previous
import jax
import jax.numpy as jnp
from jax.experimental import pallas as pl

_N_TOKENS = 16384
_INPUT_DIM = 256
_HIDDEN = 64
_K = 1024
_S = 3
_TN = 1024


def _bdot(a, b):
    return jnp.dot(a.astype(jnp.bfloat16), b,
                   preferred_element_type=jnp.float32)


def _rowsum_sq(v):
    sq = v * v
    p = sq[:, 0:8]
    for j in range(1, 8):
        p = p + sq[:, 8 * j:8 * j + 8]
    s = [p[:, k:k + 1] for k in range(8)]
    return (((s[0] + s[1]) + (s[2] + s[3]))
            + ((s[4] + s[5]) + (s[6] + s[7])))


def _colsum_sq(v):
    sq = v * v
    p = sq[0:8, :]
    for j in range(1, 8):
        p = p + sq[8 * j:8 * j + 8, :]
    s = [p[k:k + 1, :] for k in range(8)]
    return (((s[0] + s[1]) + (s[2] + s[3]))
            + ((s[4] + s[5]) + (s[6] + s[7])))


def _rvq_body(x_ref, w1_ref, b1_ref, w2_ref, b2_ref, w3_ref, b3_ref,
              w4_ref, b4_ref, cbt_ref, cbtb_ref, hml_ref,
              q_ref, idx_ref, rec_ref):
    h1 = jnp.maximum(_bdot(x_ref[...], w1_ref[...]) + b1_ref[...], 0.0)
    h = _bdot(h1, w2_ref[...]) + b2_ref[...]

    r = h
    q = jnp.zeros_like(h)
    iota = jax.lax.broadcasted_iota(jnp.int32, (_TN, _K), 1)
    for s in range(_S):
        cbt = cbt_ref[s]
        cn = jnp.sum(cbt * cbt, axis=0, keepdims=True)
        rn = jnp.sum(r * r, axis=1, keepdims=True)
        ab2 = _bdot(r, cbtb_ref[s])
        d = jnp.sqrt(jnp.maximum((rn + cn) + ab2, 0.0))
        m = jnp.min(d, axis=1, keepdims=True)
        idx = jnp.min(jnp.where(d == m, iota, _K), axis=1, keepdims=True)
        idx_ref[:, s:s + 1] = idx
        oh = (iota == idx).astype(jnp.bfloat16)
        p = jnp.dot(oh, hml_ref[s], preferred_element_type=jnp.float32)
        sel = (p[:, 0:_HIDDEN] + p[:, _HIDDEN:2 * _HIDDEN]) \
            + p[:, 2 * _HIDDEN:3 * _HIDDEN]
        q = q + sel
        r = r - sel

    q_ref[...] = q
    d1 = jnp.maximum(_bdot(q, w3_ref[...]) + b3_ref[...], 0.0)
    rec_ref[...] = _bdot(d1, w4_ref[...]) + b4_ref[...]


def kernel(x, W1, b1, W2, b2, W3, b3, W4, b4, codebooks):
    f32 = jnp.float32
    bf16 = jnp.bfloat16
    cbt = jnp.swapaxes(codebooks, 1, 2)
    cbtb = -2.0 * cbt.astype(bf16)
    hi = jax.lax.optimization_barrier(codebooks.astype(bf16))
    rem1 = codebooks - hi.astype(f32)
    mid = jax.lax.optimization_barrier(rem1.astype(bf16))
    lo = jax.lax.optimization_barrier(
        (rem1 - mid.astype(f32)).astype(bf16))
    hml = jnp.concatenate([hi, mid, lo], axis=2)

    grid = (_N_TOKENS // _TN,)
    full = lambda shape: pl.BlockSpec(shape, lambda i: (0,) * len(shape))
    q, idx, rec = pl.pallas_call(
        _rvq_body,
        grid=grid,
        in_specs=[
            pl.BlockSpec((_TN, _INPUT_DIM), lambda i: (i, 0)),
            full((_INPUT_DIM, 2 * _HIDDEN)),
            full((1, 2 * _HIDDEN)),
            full((2 * _HIDDEN, _HIDDEN)),
            full((1, _HIDDEN)),
            full((_HIDDEN, 2 * _HIDDEN)),
            full((1, 2 * _HIDDEN)),
            full((2 * _HIDDEN, _INPUT_DIM)),
            full((1, _INPUT_DIM)),
            full((_S, _HIDDEN, _K)),
            full((_S, _HIDDEN, _K)),
            full((_S, _K, 3 * _HIDDEN)),
        ],
        out_specs=[
            pl.BlockSpec((_TN, _HIDDEN), lambda i: (i, 0)),
            pl.BlockSpec((_TN, _S), lambda i: (i, 0)),
            pl.BlockSpec((_TN, _INPUT_DIM), lambda i: (i, 0)),
        ],
        out_shape=[
            jax.ShapeDtypeStruct((_N_TOKENS, _HIDDEN), f32),
            jax.ShapeDtypeStruct((_N_TOKENS, _S), jnp.int32),
            jax.ShapeDtypeStruct((_N_TOKENS, _INPUT_DIM), f32),
        ],
    )(x.astype(bf16), W1.astype(bf16), b1.reshape(1, -1), W2.astype(bf16),
      b2.reshape(1, -1), W3.astype(bf16), b3.reshape(1, -1),
      W4.astype(bf16), b4.reshape(1, -1), cbt, cbtb, hml)
    return (q, idx.T, rec)


# speedup vs baseline: 1.7933x; 1.0454x over previous
import jax
import jax.numpy as jnp
from jax.experimental import pallas as pl

_N_TOKENS = 16384
_INPUT_DIM = 256
_HIDDEN = 64
_K = 1024
_S = 3
_TN = 2048


def _bdot(a, b):
    return jnp.dot(a.astype(jnp.bfloat16), b,
                   preferred_element_type=jnp.float32)


def _rowsum_sq(v):
    sq = v * v
    p = sq[:, 0:8]
    for j in range(1, 8):
        p = p + sq[:, 8 * j:8 * j + 8]
    s = [p[:, k:k + 1] for k in range(8)]
    return (((s[0] + s[1]) + (s[2] + s[3]))
            + ((s[4] + s[5]) + (s[6] + s[7])))


def _colsum_sq(v):
    sq = v * v
    p = sq[0:8, :]
    for j in range(1, 8):
        p = p + sq[8 * j:8 * j + 8, :]
    s = [p[k:k + 1, :] for k in range(8)]
    return (((s[0] + s[1]) + (s[2] + s[3]))
            + ((s[4] + s[5]) + (s[6] + s[7])))


def _rvq_body(x_ref, w1_ref, b1_ref, w2_ref, b2_ref, w3_ref, b3_ref,
              w4_ref, b4_ref, cbt_ref, cbtb_ref, hml_ref,
              q_ref, idx_ref, rec_ref):
    h1 = jnp.maximum(_bdot(x_ref[...], w1_ref[...]) + b1_ref[...], 0.0)
    h = _bdot(h1, w2_ref[...]) + b2_ref[...]

    r = h
    q = jnp.zeros_like(h)
    iota = jax.lax.broadcasted_iota(jnp.int32, (_TN, _K), 1)
    for s in range(_S):
        cbt = cbt_ref[s]
        cn = jnp.sum(cbt * cbt, axis=0, keepdims=True)
        rn = jnp.sum(r * r, axis=1, keepdims=True)
        ab2 = _bdot(r, cbtb_ref[s])
        d = jnp.sqrt(jnp.maximum((rn + cn) + ab2, 0.0))
        m = jnp.min(d, axis=1, keepdims=True)
        idx = jnp.min(jnp.where(d == m, iota, _K), axis=1, keepdims=True)
        idx_ref[:, s:s + 1] = idx
        oh = (iota == idx).astype(jnp.bfloat16)
        p = jnp.dot(oh, hml_ref[s], preferred_element_type=jnp.float32)
        sel = (p[:, 0:_HIDDEN] + p[:, _HIDDEN:2 * _HIDDEN]) \
            + p[:, 2 * _HIDDEN:3 * _HIDDEN]
        q = q + sel
        r = r - sel

    q_ref[...] = q
    d1 = jnp.maximum(_bdot(q, w3_ref[...]) + b3_ref[...], 0.0)
    rec_ref[...] = _bdot(d1, w4_ref[...]) + b4_ref[...]


def kernel(x, W1, b1, W2, b2, W3, b3, W4, b4, codebooks):
    f32 = jnp.float32
    bf16 = jnp.bfloat16
    cbt = jnp.swapaxes(codebooks, 1, 2)
    cbtb = -2.0 * cbt.astype(bf16)
    hi = jax.lax.optimization_barrier(codebooks.astype(bf16))
    rem1 = codebooks - hi.astype(f32)
    mid = jax.lax.optimization_barrier(rem1.astype(bf16))
    lo = jax.lax.optimization_barrier(
        (rem1 - mid.astype(f32)).astype(bf16))
    hml = jnp.concatenate([hi, mid, lo], axis=2)

    grid = (_N_TOKENS // _TN,)
    full = lambda shape: pl.BlockSpec(shape, lambda i: (0,) * len(shape))
    q, idx, rec = pl.pallas_call(
        _rvq_body,
        grid=grid,
        in_specs=[
            pl.BlockSpec((_TN, _INPUT_DIM), lambda i: (i, 0)),
            full((_INPUT_DIM, 2 * _HIDDEN)),
            full((1, 2 * _HIDDEN)),
            full((2 * _HIDDEN, _HIDDEN)),
            full((1, _HIDDEN)),
            full((_HIDDEN, 2 * _HIDDEN)),
            full((1, 2 * _HIDDEN)),
            full((2 * _HIDDEN, _INPUT_DIM)),
            full((1, _INPUT_DIM)),
            full((_S, _HIDDEN, _K)),
            full((_S, _HIDDEN, _K)),
            full((_S, _K, 3 * _HIDDEN)),
        ],
        out_specs=[
            pl.BlockSpec((_TN, _HIDDEN), lambda i: (i, 0)),
            pl.BlockSpec((_TN, _S), lambda i: (i, 0)),
            pl.BlockSpec((_TN, _INPUT_DIM), lambda i: (i, 0)),
        ],
        out_shape=[
            jax.ShapeDtypeStruct((_N_TOKENS, _HIDDEN), f32),
            jax.ShapeDtypeStruct((_N_TOKENS, _S), jnp.int32),
            jax.ShapeDtypeStruct((_N_TOKENS, _INPUT_DIM), f32),
        ],
    )(x.astype(bf16), W1.astype(bf16), b1.reshape(1, -1), W2.astype(bf16),
      b2.reshape(1, -1), W3.astype(bf16), b3.reshape(1, -1),
      W4.astype(bf16), b4.reshape(1, -1), cbt, cbtb, hml)
    return (q, idx.T, rec)
